# baseline (speedup 1.0000x reference)
import functools

import jax
import jax.numpy as jnp
from jax import lax
from jax.experimental import pallas as pl
from jax.experimental.pallas import tpu as pltpu
from jax.experimental.pallas import tpu_sc as plsc

_N = 10000
_D = 128
_NC = 2
_NS = 16
_CHUNK = 128
_NPAD = 10240
_ROWS = _NPAD // _NS


def _sc_mesh():
    return plsc.VectorSubcoreMesh(core_axis_name="c", subcore_axis_name="s")


_NBUF = 4


def _make_agg(C):
    assert C % _NBUF == 0

    @functools.partial(
        pl.kernel,
        out_type=jax.ShapeDtypeStruct((_NC, _NPAD, _D), jnp.float32),
        mesh=_sc_mesh(),
        scratch_types=[
            pltpu.VMEM((C, _CHUNK), jnp.int32),
            pltpu.VMEM((C, _CHUNK), jnp.int32),
            pltpu.VMEM((_CHUNK, _D), jnp.float32),
            pltpu.VMEM_SHARED((_NPAD, _D), jnp.float32),
        ],
    )
    def agg(z_hbm, src_hbm, dst_hbm, zeros_hbm, out_hbm, srcv, dstv, rowsv,
            acc):
        c = lax.axis_index("c")
        s = lax.axis_index("s")
        pltpu.sync_copy(src_hbm.at[c, s], srcv)
        pltpu.sync_copy(dst_hbm.at[c, s], dstv)
        pltpu.sync_copy(zeros_hbm.at[pl.ds(s * _ROWS, _ROWS)],
                        acc.at[pl.ds(s * _ROWS, _ROWS)])
        plsc.subcore_barrier()

        @pl.loop(0, C)
        def _(j):
            pltpu.sync_copy(z_hbm.at[srcv.at[j]], rowsv)
            pltpu.sync_copy(rowsv, acc.at[dstv.at[j]], add=True)

        plsc.subcore_barrier()
        pltpu.sync_copy(acc.at[pl.ds(s * _ROWS, _ROWS)],
                        out_hbm.at[c, pl.ds(s * _ROWS, _ROWS)])

    return agg


def _tc_first(x, W1, cntp):
    def body(x_ref, w_ref, cnt_ref, out_ref):
        cnt = cnt_ref[0] + cnt_ref[1]
        dinv = lax.rsqrt(cnt + 1.0)[:_N, 0:1]
        h = jnp.dot(x_ref[...], w_ref[...], preferred_element_type=jnp.float32)
        out_ref[...] = h * dinv

    return pl.pallas_call(
        body, out_shape=jax.ShapeDtypeStruct((_N, _D), jnp.float32)
    )(x, W1, cntp)


def _tc_mid(Sp, zt, cntp, b, W2):
    def body(sp_ref, zt_ref, cnt_ref, b_ref, w_ref, out_ref):
        cnt = cnt_ref[0] + cnt_ref[1]
        dinv = lax.rsqrt(cnt + 1.0)[:_N, 0:1]
        S = sp_ref[0, :_N, :] + sp_ref[1, :_N, :] + zt_ref[...]
        h = jnp.maximum(S * dinv + b_ref[...], 0.0)
        out_ref[...] = jnp.dot(
            h, w_ref[...], preferred_element_type=jnp.float32) * dinv

    return pl.pallas_call(
        body, out_shape=jax.ShapeDtypeStruct((_N, _D), jnp.float32)
    )(Sp, zt, cntp, b, W2)


def _tc_last(Sp, zt, cntp, b, Wl, bl):
    def body(sp_ref, zt_ref, cnt_ref, b_ref, wl_ref, bl_ref, out_ref):
        cnt = cnt_ref[0] + cnt_ref[1]
        dinv = lax.rsqrt(cnt + 1.0)[:_N, 0:1]
        S = sp_ref[0, :_N, :] + sp_ref[1, :_N, :] + zt_ref[...]
        h = jnp.maximum(S * dinv + b_ref[...], 0.0)
        out_ref[...] = jnp.dot(
            h, wl_ref[...], preferred_element_type=jnp.float32) + bl_ref[...]

    return pl.pallas_call(
        body, out_shape=jax.ShapeDtypeStruct((_N, Wl.shape[1]), jnp.float32)
    )(Sp, zt, cntp, b, Wl, bl)


def kernel(x, adjacency, W1, b1, W2, b2, Wl, bl):
    E = adjacency.shape[1]
    per = _NC * _NS * _CHUNK
    C = -(-E // per)
    C = -(-C // _NBUF) * _NBUF
    pad = C * per - E

    src = adjacency[0].astype(jnp.int32)
    dst = adjacency[1].astype(jnp.int32)
    src = jnp.concatenate([src, jnp.zeros((pad,), jnp.int32)])
    dst = jnp.concatenate([dst, jnp.full((pad,), _N, jnp.int32)])
    src4 = src.reshape(_NC, _NS, C, _CHUNK)
    dst4 = dst.reshape(_NC, _NS, C, _CHUNK)

    zerosD = jnp.zeros((_NPAD, _D), jnp.float32)
    onesZ = jnp.ones((_N, _D), jnp.float32)

    agg = _make_agg(C)

    cntp = agg(onesZ, jnp.zeros_like(src4), dst4, zerosD)
    z1t = _tc_first(x, W1, cntp)
    S1p = agg(z1t, src4, dst4, zerosD)
    z2t = _tc_mid(S1p, z1t, cntp, b1.reshape(1, _D), W2)
    S2p = agg(z2t, src4, dst4, zerosD)
    out = _tc_last(S2p, z2t, cntp, b2.reshape(1, _D), Wl, bl.reshape(1, 2))
    return out


# speedup vs baseline: 12.4152x; 12.4152x over previous
import functools

import jax
import jax.numpy as jnp
from jax import lax
from jax.experimental import pallas as pl
from jax.experimental.pallas import tpu as pltpu
from jax.experimental.pallas import tpu_sc as plsc

_N = 10000
_D = 128
_NC = 2
_NS = 16
_CHUNK = 128
_NPAD = 10240
_ROWS = _NPAD // _NS


def _sc_mesh():
    return plsc.VectorSubcoreMesh(core_axis_name="c", subcore_axis_name="s")


def _make_hist(C):
    @functools.partial(
        pl.kernel,
        out_type=jax.ShapeDtypeStruct((_NC, _NPAD, _D), jnp.float32),
        mesh=_sc_mesh(),
        scratch_types=[
            pltpu.VMEM((C, _CHUNK), jnp.int32),
            pltpu.VMEM((_CHUNK, _D), jnp.float32),
            pltpu.VMEM_SHARED((_NPAD, _D), jnp.float32),
        ],
    )
    def hist(dst_hbm, ones_hbm, zeros_hbm, out_hbm, dstv, onesv, acc):
        c = lax.axis_index("c")
        s = lax.axis_index("s")
        pltpu.sync_copy(dst_hbm.at[c, s], dstv)
        pltpu.sync_copy(ones_hbm, onesv)
        pltpu.sync_copy(zeros_hbm.at[pl.ds(s * _ROWS, _ROWS)],
                        acc.at[pl.ds(s * _ROWS, _ROWS)])
        plsc.subcore_barrier()

        @pl.loop(0, C)
        def _(j):
            pltpu.sync_copy(onesv, acc.at[dstv.at[j]], add=True)

        plsc.subcore_barrier()
        pltpu.sync_copy(acc.at[pl.ds(s * _ROWS, _ROWS)],
                        out_hbm.at[c, pl.ds(s * _ROWS, _ROWS)])

    return hist


_IB = 8


def _make_agg(C):
    assert C % _IB == 0

    @functools.partial(
        pl.kernel,
        out_type=jax.ShapeDtypeStruct((_NC, _NPAD, _D), jnp.float32),
        mesh=_sc_mesh(),
        scratch_types=[
            pltpu.VMEM((_IB, _CHUNK), jnp.int32),
            pltpu.VMEM((_IB, _CHUNK), jnp.int32),
            pltpu.VMEM((_CHUNK, _D), jnp.float32),
            pltpu.VMEM((_CHUNK, _D), jnp.float32),
            pltpu.VMEM_SHARED((_NPAD, _D), jnp.float32),
        ],
    )
    def agg(z_hbm, src_hbm, dst_hbm, zeros_hbm, out_hbm, srcv, dstv, buf0, buf1,
            acc):
        c = lax.axis_index("c")
        s = lax.axis_index("s")
        pltpu.sync_copy(zeros_hbm.at[pl.ds(s * _ROWS, _ROWS)],
                        acc.at[pl.ds(s * _ROWS, _ROWS)])
        plsc.subcore_barrier()

        bufs = (buf0, buf1)

        def pipelined(sem0):
            @pl.loop(0, C, step=_IB)
            def _(jb):
                pltpu.sync_copy(src_hbm.at[c, s, pl.ds(jb, _IB)], srcv)
                pltpu.sync_copy(dst_hbm.at[c, s, pl.ds(jb, _IB)], dstv)
                descs = [None] * _IB
                descs[0] = pltpu.async_copy(z_hbm.at[srcv.at[0]], bufs[0],
                                            sem0)
                for k in range(_IB):
                    if k + 1 < _IB:
                        descs[k + 1] = pltpu.async_copy(
                            z_hbm.at[srcv.at[k + 1]], bufs[(k + 1) % 2], sem0)
                    descs[k].wait()
                    pltpu.sync_copy(bufs[k % 2], acc.at[dstv.at[k]], add=True)

        pl.run_scoped(pipelined, pltpu.SemaphoreType.DMA)

        plsc.subcore_barrier()
        pltpu.sync_copy(acc.at[pl.ds(s * _ROWS, _ROWS)],
                        out_hbm.at[c, pl.ds(s * _ROWS, _ROWS)])

    return agg


def _tc_first(x, W1, cntp):
    def body(x_ref, w_ref, cnt_ref, out_ref):
        cnt = cnt_ref[0] + cnt_ref[1]
        dinv = lax.rsqrt(cnt + 1.0)[:_N, 0:1]
        h = jnp.dot(x_ref[...], w_ref[...], preferred_element_type=jnp.float32)
        out_ref[...] = h * dinv

    return pl.pallas_call(
        body, out_shape=jax.ShapeDtypeStruct((_N, _D), jnp.float32)
    )(x, W1, cntp)


def _tc_mid(Sp, zt, cntp, b, W2):
    def body(sp_ref, zt_ref, cnt_ref, b_ref, w_ref, out_ref):
        cnt = cnt_ref[0] + cnt_ref[1]
        dinv = lax.rsqrt(cnt + 1.0)[:_N, 0:1]
        S = sp_ref[0, :_N, :] + sp_ref[1, :_N, :] + zt_ref[...]
        h = jnp.maximum(S * dinv + b_ref[...], 0.0)
        out_ref[...] = jnp.dot(
            h, w_ref[...], preferred_element_type=jnp.float32) * dinv

    return pl.pallas_call(
        body, out_shape=jax.ShapeDtypeStruct((_N, _D), jnp.float32)
    )(Sp, zt, cntp, b, W2)


def _tc_last(Sp, zt, cntp, b, Wl, bl):
    def body(sp_ref, zt_ref, cnt_ref, b_ref, wl_ref, bl_ref, out_ref):
        cnt = cnt_ref[0] + cnt_ref[1]
        dinv = lax.rsqrt(cnt + 1.0)[:_N, 0:1]
        S = sp_ref[0, :_N, :] + sp_ref[1, :_N, :] + zt_ref[...]
        h = jnp.maximum(S * dinv + b_ref[...], 0.0)
        out_ref[...] = jnp.dot(
            h, wl_ref[...], preferred_element_type=jnp.float32) + bl_ref[...]

    return pl.pallas_call(
        body, out_shape=jax.ShapeDtypeStruct((_N, Wl.shape[1]), jnp.float32)
    )(Sp, zt, cntp, b, Wl, bl)


def kernel(x, adjacency, W1, b1, W2, b2, Wl, bl):
    E = adjacency.shape[1]
    per = _NC * _NS * _CHUNK
    C = -(-E // per)
    C = -(-C // _IB) * _IB
    pad = C * per - E

    src = adjacency[0].astype(jnp.int32)
    dst = adjacency[1].astype(jnp.int32)
    src = jnp.concatenate([src, jnp.zeros((pad,), jnp.int32)])
    dst = jnp.concatenate([dst, jnp.full((pad,), _N, jnp.int32)])
    src4 = src.reshape(_NC, _NS, C, _CHUNK)
    dst4 = dst.reshape(_NC, _NS, C, _CHUNK)

    zerosD = jnp.zeros((_NPAD, _D), jnp.float32)
    onesD = jnp.ones((_CHUNK, _D), jnp.float32)

    hist = _make_hist(C)
    agg = _make_agg(C)

    cntp = hist(dst4, onesD, zerosD)
    z1t = _tc_first(x, W1, cntp)
    S1p = agg(z1t, src4, dst4, zerosD)
    z2t = _tc_mid(S1p, z1t, cntp, b1.reshape(1, _D), W2)
    S2p = agg(z2t, src4, dst4, zerosD)
    out = _tc_last(S2p, z2t, cntp, b2.reshape(1, _D), Wl, bl.reshape(1, 2))
    return out


# speedup vs baseline: 27.2722x; 2.1967x over previous
import functools

import jax
import jax.numpy as jnp
from jax import lax
from jax.experimental import pallas as pl
from jax.experimental.pallas import tpu as pltpu
from jax.experimental.pallas import tpu_sc as plsc

_N = 10000
_D = 128
_NC = 2
_NS = 16
_CHUNK = 128
_NPAD = 10240
_ROWS = _NPAD // _NS


def _sc_mesh():
    return plsc.VectorSubcoreMesh(core_axis_name="c", subcore_axis_name="s")


def _make_hist(C):
    @functools.partial(
        pl.kernel,
        out_type=jax.ShapeDtypeStruct((_NC, _NPAD, _D), jnp.float32),
        mesh=_sc_mesh(),
        scratch_types=[
            pltpu.VMEM((C, _CHUNK), jnp.int32),
            pltpu.VMEM((_CHUNK, _D), jnp.float32),
            pltpu.VMEM_SHARED((_NPAD, _D), jnp.float32),
        ],
    )
    def hist(dst_hbm, ones_hbm, zeros_hbm, out_hbm, dstv, onesv, acc):
        c = lax.axis_index("c")
        s = lax.axis_index("s")
        pltpu.sync_copy(dst_hbm.at[c, s], dstv)
        pltpu.sync_copy(ones_hbm, onesv)
        pltpu.sync_copy(zeros_hbm.at[pl.ds(s * _ROWS, _ROWS)],
                        acc.at[pl.ds(s * _ROWS, _ROWS)])
        plsc.subcore_barrier()

        @pl.loop(0, C)
        def _(j):
            pltpu.sync_copy(onesv, acc.at[dstv.at[j]], add=True)

        plsc.subcore_barrier()
        pltpu.sync_copy(acc.at[pl.ds(s * _ROWS, _ROWS)],
                        out_hbm.at[c, pl.ds(s * _ROWS, _ROWS)])

    return hist


_IB = 8


def _make_agg(C):
    assert C % _IB == 0

    @functools.partial(
        pl.kernel,
        out_type=jax.ShapeDtypeStruct((_NC, _NPAD, _D), jnp.float32),
        mesh=_sc_mesh(),
        scratch_types=[
            pltpu.VMEM((C, _CHUNK), jnp.int32),
            pltpu.VMEM((C, _CHUNK), jnp.int32),
            pltpu.VMEM((_CHUNK, _D), jnp.float32),
            pltpu.VMEM_SHARED((_NPAD, _D), jnp.float32),
        ],
    )
    def agg(z_hbm, src_hbm, dst_hbm, zeros_hbm, out_hbm, srcv, dstv, rowsv,
            acc):
        c = lax.axis_index("c")
        s = lax.axis_index("s")
        pltpu.sync_copy(src_hbm.at[c, s], srcv)
        pltpu.sync_copy(dst_hbm.at[c, s], dstv)
        pltpu.sync_copy(zeros_hbm.at[pl.ds(s * _ROWS, _ROWS)],
                        acc.at[pl.ds(s * _ROWS, _ROWS)])
        plsc.subcore_barrier()

        @pl.loop(0, C)
        def _(j):
            pltpu.sync_copy(z_hbm.at[srcv.at[j]], rowsv)
            pltpu.sync_copy(rowsv, acc.at[dstv.at[j]], add=True)

        plsc.subcore_barrier()
        pltpu.sync_copy(acc.at[pl.ds(s * _ROWS, _ROWS)],
                        out_hbm.at[c, pl.ds(s * _ROWS, _ROWS)])

    return agg


def _tc_first(x, W1, cntp):
    def body(x_ref, w_ref, cnt_ref, out_ref):
        cnt = cnt_ref[0] + cnt_ref[1]
        dinv = lax.rsqrt(cnt + 1.0)[:_N, 0:1]
        h = jnp.dot(x_ref[...], w_ref[...], preferred_element_type=jnp.float32)
        out_ref[...] = h * dinv

    return pl.pallas_call(
        body, out_shape=jax.ShapeDtypeStruct((_N, _D), jnp.float32)
    )(x, W1, cntp)


def _tc_mid(Sp, zt, cntp, b, W2):
    def body(sp_ref, zt_ref, cnt_ref, b_ref, w_ref, out_ref):
        cnt = cnt_ref[0] + cnt_ref[1]
        dinv = lax.rsqrt(cnt + 1.0)[:_N, 0:1]
        S = sp_ref[0, :_N, :] + sp_ref[1, :_N, :] + zt_ref[...]
        h = jnp.maximum(S * dinv + b_ref[...], 0.0)
        out_ref[...] = jnp.dot(
            h, w_ref[...], preferred_element_type=jnp.float32) * dinv

    return pl.pallas_call(
        body, out_shape=jax.ShapeDtypeStruct((_N, _D), jnp.float32)
    )(Sp, zt, cntp, b, W2)


def _tc_last(Sp, zt, cntp, b, Wl, bl):
    def body(sp_ref, zt_ref, cnt_ref, b_ref, wl_ref, bl_ref, out_ref):
        cnt = cnt_ref[0] + cnt_ref[1]
        dinv = lax.rsqrt(cnt + 1.0)[:_N, 0:1]
        S = sp_ref[0, :_N, :] + sp_ref[1, :_N, :] + zt_ref[...]
        h = jnp.maximum(S * dinv + b_ref[...], 0.0)
        out_ref[...] = jnp.dot(
            h, wl_ref[...], preferred_element_type=jnp.float32) + bl_ref[...]

    return pl.pallas_call(
        body, out_shape=jax.ShapeDtypeStruct((_N, Wl.shape[1]), jnp.float32)
    )(Sp, zt, cntp, b, Wl, bl)


def kernel(x, adjacency, W1, b1, W2, b2, Wl, bl):
    E = adjacency.shape[1]
    per = _NC * _NS * _CHUNK
    C = -(-E // per)
    C = -(-C // _IB) * _IB
    pad = C * per - E

    src = adjacency[0].astype(jnp.int32)
    dst = adjacency[1].astype(jnp.int32)
    fill = jnp.arange(pad, dtype=jnp.int32)
    src = jnp.concatenate([src, (fill * 37) % _N])
    dst = jnp.concatenate([dst, _N + (fill % (_NPAD - _N))])
    src4 = src.reshape(_NC, _NS, C, _CHUNK)
    dst4 = dst.reshape(_NC, _NS, C, _CHUNK)

    zerosD = jnp.zeros((_NPAD, _D), jnp.float32)
    onesD = jnp.ones((_CHUNK, _D), jnp.float32)

    hist = _make_hist(C)
    agg = _make_agg(C)

    cntp = hist(dst4, onesD, zerosD)
    z1t = _tc_first(x, W1, cntp)
    S1p = agg(z1t, src4, dst4, zerosD)
    z2t = _tc_mid(S1p, z1t, cntp, b1.reshape(1, _D), W2)
    S2p = agg(z2t, src4, dst4, zerosD)
    out = _tc_last(S2p, z2t, cntp, b2.reshape(1, _D), Wl, bl.reshape(1, 2))
    return out
